# Initial kernel scaffold; baseline (speedup 1.0000x reference)
import jax
import jax.numpy as jnp
from jax.experimental import pallas as pl


def kernel(input, adj, W, a):
    raise NotImplementedError("write your pallas kernel here")


# speedup vs baseline: 1.6321x; 1.6321x over previous
import functools

import jax
import jax.numpy as jnp
from jax.experimental import pallas as pl

N = 4096
IN_F = 128
OUT_F = 128
ALPHA = 0.2
BM = 512


def _proj_kernel(x_ref, w_ref, a1_ref, a2_ref, h_ref, src_ref, dst_ref):
    h = jnp.dot(x_ref[...], w_ref[...], preferred_element_type=jnp.float32)
    h_ref[...] = h
    src_ref[...] = jnp.dot(h, a1_ref[...], preferred_element_type=jnp.float32)
    dst_ref[...] = jnp.dot(h, a2_ref[...], preferred_element_type=jnp.float32)


def _attn_kernel(adj_ref, h_ref, src_ref, dstt_ref, out_ref):
    logits = src_ref[...] + dstt_ref[...]
    logits = jnp.where(logits > 0, logits, ALPHA * logits)
    logits = jnp.where(adj_ref[...] > 0, logits, jnp.float32(-9e15))
    m = jnp.max(logits, axis=1, keepdims=True)
    e = jnp.exp(logits - m)
    s = jnp.sum(e, axis=1, keepdims=True)
    hp = jnp.dot(e, h_ref[...], preferred_element_type=jnp.float32) / s
    out_ref[...] = jnp.where(hp > 0, hp, jnp.exp(jnp.minimum(hp, 0.0)) - 1.0)


@jax.jit
def kernel(input, adj, W, a):
    a1 = a[:OUT_F].reshape(IN_F, 1)
    a2 = a[OUT_F:].reshape(IN_F, 1)
    h, src, dst = pl.pallas_call(
        _proj_kernel,
        out_shape=(
            jax.ShapeDtypeStruct((N, OUT_F), jnp.float32),
            jax.ShapeDtypeStruct((N, 1), jnp.float32),
            jax.ShapeDtypeStruct((N, 1), jnp.float32),
        ),
    )(input, W, a1, a2)
    dstt = dst.reshape(1, N)

    out = pl.pallas_call(
        _attn_kernel,
        grid=(N // BM,),
        in_specs=[
            pl.BlockSpec((BM, N), lambda i: (i, 0)),
            pl.BlockSpec((N, OUT_F), lambda i: (0, 0)),
            pl.BlockSpec((BM, 1), lambda i: (i, 0)),
            pl.BlockSpec((1, N), lambda i: (0, 0)),
        ],
        out_specs=pl.BlockSpec((BM, OUT_F), lambda i: (i, 0)),
        out_shape=jax.ShapeDtypeStruct((N, OUT_F), jnp.float32),
    )(adj, h, src, dstt)
    return out


# speedup vs baseline: 1.8861x; 1.1556x over previous
import jax
import jax.numpy as jnp
from jax.experimental import pallas as pl

N = 4096
IN_F = 128
OUT_F = 128
ALPHA = 0.2
BM = 512
LOG2E = 1.4426950408889634


def _proj_kernel(x_ref, w_ref, a1_ref, a2_ref,
                 hb_ref, srcs_ref, dsts_ref, dmax_ref, meanh_ref):
    h = jnp.dot(x_ref[...], w_ref[...], preferred_element_type=jnp.float32)
    hb_ref[...] = h.astype(jnp.bfloat16)
    meanh_ref[...] = jnp.mean(h, axis=0, keepdims=True)
    srcs_ref[...] = jnp.dot(h, a1_ref[...], preferred_element_type=jnp.float32) * LOG2E
    dsts = jnp.dot(h, a2_ref[...], preferred_element_type=jnp.float32) * LOG2E
    dsts_ref[...] = dsts
    dmax_ref[...] = jnp.max(dsts).reshape(1, 1)


def _attn_kernel(adj_ref, hb_ref, srcs_ref, dstts_ref, dmax_ref, meanh_ref,
                 out_ref):
    srcs = srcs_ref[...]
    t = srcs + dmax_ref[0, 0]
    m = jnp.maximum(t, ALPHA * t)
    l0 = srcs + dstts_ref[...]
    lk = jnp.maximum(l0, ALPHA * l0)
    e = jnp.exp2(lk - m) * adj_ref[...]
    s = jnp.sum(e, axis=1, keepdims=True)
    hp = jnp.dot(e.astype(jnp.bfloat16), hb_ref[...],
                 preferred_element_type=jnp.float32)
    s_safe = jnp.where(s > 0, s, 1.0)
    hp = jnp.where(s > 0, hp / s_safe, meanh_ref[...])
    out_ref[...] = jnp.where(hp > 0, hp, jnp.exp(jnp.minimum(hp, 0.0)) - 1.0)


@jax.jit
def kernel(input, adj, W, a):
    a1 = a[:OUT_F].reshape(IN_F, 1)
    a2 = a[OUT_F:].reshape(IN_F, 1)
    hb, srcs, dsts, dmax, meanh = pl.pallas_call(
        _proj_kernel,
        out_shape=(
            jax.ShapeDtypeStruct((N, OUT_F), jnp.bfloat16),
            jax.ShapeDtypeStruct((N, 1), jnp.float32),
            jax.ShapeDtypeStruct((N, 1), jnp.float32),
            jax.ShapeDtypeStruct((1, 1), jnp.float32),
            jax.ShapeDtypeStruct((1, OUT_F), jnp.float32),
        ),
    )(input, W, a1, a2)
    dstts = dsts.reshape(1, N)

    out = pl.pallas_call(
        _attn_kernel,
        grid=(N // BM,),
        in_specs=[
            pl.BlockSpec((BM, N), lambda i: (i, 0)),
            pl.BlockSpec((N, OUT_F), lambda i: (0, 0)),
            pl.BlockSpec((BM, 1), lambda i: (i, 0)),
            pl.BlockSpec((1, N), lambda i: (0, 0)),
            pl.BlockSpec((1, 1), lambda i: (0, 0)),
            pl.BlockSpec((1, OUT_F), lambda i: (0, 0)),
        ],
        out_specs=pl.BlockSpec((BM, OUT_F), lambda i: (i, 0)),
        out_shape=jax.ShapeDtypeStruct((N, OUT_F), jnp.float32),
    )(adj, hb, srcs, dstts, dmax, meanh)
    return out
